# baseline (speedup 1.0000x reference)
import functools

import jax
import jax.numpy as jnp
from jax import lax
from jax.experimental import pallas as pl
from jax.experimental.pallas import tpu as pltpu
from jax.experimental.pallas import tpu_sc as plsc

N = 10000
E = 320000
D = 128
H = 128
NA = 16

NC = 2
NS = 16
NW = NC * NS
E2 = E // 2
EW = E2 // NW
CH = 40
NCH = EW // CH
DP = 4
NP = 10240
ZR = NP // NS

BE = 2000
GRID_E2 = E2 // BE
BN = 2000
GRID_N = N // BN


def _psi(p):
    return jnp.sign(p) * jnp.log(jnp.abs(p) + 1)


def _pack2(lo_f32, hi_f32):
    lo = lax.bitcast_convert_type(lo_f32.astype(jnp.bfloat16),
                                  jnp.uint16).astype(jnp.uint32)
    hi = lax.bitcast_convert_type(hi_f32.astype(jnp.bfloat16),
                                  jnp.uint16).astype(jnp.uint32)
    return lax.bitcast_convert_type(lo | (hi << 16), jnp.int32)


def _ab_body(h_ref, x_ref, wa_ref, wb_ref, ta_ref, tb_ref):
    a = jnp.dot(h_ref[...], wa_ref[...], preferred_element_type=jnp.float32)
    b = jnp.dot(h_ref[...], wb_ref[...], preferred_element_type=jnp.float32)
    x = x_ref[...]
    xpad = jnp.concatenate([x, jnp.zeros((x.shape[0], D - 4), jnp.float32)],
                           axis=1)
    ta_ref[...] = _pack2(a, xpad)
    tb_ref[...] = _pack2(b, xpad)


def _gather_body(ebase, i_hbm, j_hbm, ta_hbm, tb_hbm, gi_out, gj_out,
                 idxi, idxj, bufa, bufb, isem, gsem, wsem):
    wid = lax.axis_index("s") * NC + lax.axis_index("c")
    base = ebase + wid * EW

    pltpu.make_async_copy(i_hbm.at[pl.ds(base, EW)], idxi, isem).start()
    pltpu.make_async_copy(j_hbm.at[pl.ds(base, EW)], idxj, isem).start()
    pltpu.make_async_copy(i_hbm.at[pl.ds(base, EW)], idxi, isem).wait()
    pltpu.make_async_copy(j_hbm.at[pl.ds(base, EW)], idxj, isem).wait()

    def gat_descs(t):
        b = lax.rem(t, DP)
        o = t * CH
        return [
            pltpu.make_async_copy(ta_hbm.at[idxi.at[pl.ds(o, CH)]],
                                  bufa.at[b], gsem.at[b]),
            pltpu.make_async_copy(tb_hbm.at[idxj.at[pl.ds(o, CH)]],
                                  bufb.at[b], gsem.at[b]),
        ]

    obase = wid * EW

    def wr_descs(t):
        b = lax.rem(t, DP)
        s = obase + t * CH
        return [
            pltpu.make_async_copy(bufa.at[b], gi_out.at[pl.ds(s, CH)], wsem.at[b]),
            pltpu.make_async_copy(bufb.at[b], gj_out.at[pl.ds(s, CH)], wsem.at[b]),
        ]

    def fire(descs):
        for d in descs:
            d.start()

    def drain(descs):
        for d in descs:
            d.wait()

    fire(gat_descs(0))
    fire(gat_descs(1))
    fire(gat_descs(2))

    def body(t, carry):
        drain(gat_descs(t))
        fire(wr_descs(t))

        @pl.when(t + 3 < NCH)
        def _():
            @pl.when(t >= 1)
            def _():
                drain(wr_descs(t - 1))

            fire(gat_descs(t + 3))

        return carry

    lax.fori_loop(0, NCH, body, 0)
    drain(wr_descs(NCH - 4))
    drain(wr_descs(NCH - 3))
    drain(wr_descs(NCH - 2))
    drain(wr_descs(NCH - 1))


def _sc_gather(i_idx, j_idx, ta, tb, ebase):
    mesh = plsc.VectorSubcoreMesh(core_axis_name="c", subcore_axis_name="s")
    return pl.kernel(
        functools.partial(_gather_body, ebase),
        out_type=[
            jax.ShapeDtypeStruct((E2, D), jnp.int32),
            jax.ShapeDtypeStruct((E2, D), jnp.int32),
        ],
        mesh=mesh,
        scratch_types=[
            pltpu.VMEM((EW,), jnp.int32),
            pltpu.VMEM((EW,), jnp.int32),
            pltpu.VMEM((DP, CH, D), jnp.int32),
            pltpu.VMEM((DP, CH, D), jnp.int32),
            pltpu.SemaphoreType.DMA,
            pltpu.SemaphoreType.DMA((DP,)),
            pltpu.SemaphoreType.DMA((DP,)),
        ],
    )(i_idx, j_idx, ta, tb)


def _unpack_lo(v_u32):
    return lax.bitcast_convert_type(v_u32 << 16, jnp.float32)


def _unpack_hi(v_u32):
    return lax.bitcast_convert_type(v_u32 & jnp.uint32(0xFFFF0000),
                                    jnp.float32)


def _e1_body(gi_ref, gj_ref, wnd_ref, z1_ref, xd_ref, ssum_ref, ssq_ref):
    gi = lax.bitcast_convert_type(gi_ref[...], jnp.uint32)
    gj = lax.bitcast_convert_type(gj_ref[...], jnp.uint32)
    ai = _unpack_lo(gi)
    bj = _unpack_lo(gj)
    xi = _unpack_hi(gi[:, :4])
    xj = _unpack_hi(gj[:, :4])
    xd = xi - xj
    psq = xd * xd
    norms = _psi(2.0 * psq[:, :1] - jnp.sum(psq, axis=1, keepdims=True))
    dsq = xi * xj
    dots = _psi(2.0 * dsq[:, :1] - jnp.sum(dsq, axis=1, keepdims=True))
    wnd = wnd_ref[...]
    z1 = ai + bj + norms * wnd[:1] + dots * wnd[1:2]
    z1_ref[...] = z1.astype(jnp.bfloat16)
    xd_ref[...] = xd

    @pl.when(pl.program_id(0) == 0)
    def _():
        ssum_ref[...] = jnp.zeros_like(ssum_ref)
        ssq_ref[...] = jnp.zeros_like(ssq_ref)

    ssum_ref[...] += jnp.sum(z1, axis=0, keepdims=True)
    ssq_ref[...] += jnp.sum(z1 * z1, axis=0, keepdims=True)


def _e2_body(z1_ref, xd_ref, g_ref, b_ref, we2_ref, be2_ref, wm_ref,
             bm_ref, wx1_ref, bx1_ref, wx2_ref, m_ref, mfull_ref, tr_ref):
    z1 = z1_ref[...].astype(jnp.float32)
    z = jnp.maximum(z1 * g_ref[...] + b_ref[...], 0.0)
    out = jnp.maximum(
        jnp.dot(z.astype(jnp.bfloat16), we2_ref[...].astype(jnp.bfloat16),
                preferred_element_type=jnp.float32)
        + be2_ref[...], 0.0)
    w = jax.nn.sigmoid(
        jnp.dot(out, wm_ref[...], preferred_element_type=jnp.float32)
        + bm_ref[...])
    m = out * w
    m_ref[...] = m
    mfull_ref[...] = m
    t = jnp.maximum(
        jnp.dot(m.astype(jnp.bfloat16), wx1_ref[...].astype(jnp.bfloat16),
                preferred_element_type=jnp.float32)
        + bx1_ref[...], 0.0)
    phix = jnp.dot(t, wx2_ref[...], preferred_element_type=jnp.float32)
    trans = jnp.clip(xd_ref[...] * phix, -100.0, 100.0)
    rows = trans.shape[0]
    tr_ref[...] = jnp.concatenate(
        [trans, jnp.ones((rows, 1), jnp.float32),
         jnp.zeros((rows, 3), jnp.float32)], axis=1)


def _e2_body_alias(z1_ref, xd_ref, g_ref, b_ref, we2_ref, be2_ref, wm_ref,
                   bm_ref, wx1_ref, bx1_ref, wx2_ref, mprev_ref, m_ref,
                   mfull_ref, tr_ref):
    del mprev_ref
    _e2_body(z1_ref, xd_ref, g_ref, b_ref, we2_ref, be2_ref, wm_ref,
             bm_ref, wx1_ref, bx1_ref, wx2_ref, m_ref, mfull_ref, tr_ref)


def _scatter_body(w_, ebase, voff, v_hbm, i_hbm, zrows_hbm, out_hbm,
                  idx, vbuf, acc_sh, lsem, ssem):
    cid = lax.axis_index("c")
    sid = lax.axis_index("s")
    wid = sid * NC + cid
    base = ebase + wid * EW
    vbase = voff + wid * EW
    rows0 = sid * ZR

    pltpu.sync_copy(zrows_hbm, acc_sh.at[pl.ds(rows0, ZR)])
    plsc.subcore_barrier()

    def ld_descs(t):
        b = lax.rem(t, DP)
        s = base + t * CH
        sv = vbase + t * CH
        return [
            pltpu.make_async_copy(i_hbm.at[pl.ds(s, CH)], idx.at[b], lsem.at[b]),
            pltpu.make_async_copy(v_hbm.at[pl.ds(sv, CH)], vbuf.at[b], lsem.at[b]),
        ]

    def sc_descs(t):
        b = lax.rem(t, DP)
        return [
            pltpu.make_async_copy(vbuf.at[b], acc_sh.at[idx.at[b]], ssem.at[b]),
        ]

    for t0 in (0, 1, 2):
        for d in ld_descs(t0):
            d.start()

    def body(t, carry):
        for d in ld_descs(t):
            d.wait()
        for d in sc_descs(t):
            d.start(add=True)

        @pl.when(t + 3 < NCH)
        def _():
            @pl.when(t >= 1)
            def _():
                for d in sc_descs(t - 1):
                    d.wait()

            for d in ld_descs(t + 3):
                d.start()

        return carry

    lax.fori_loop(0, NCH, body, 0)
    for t0 in (NCH - 4, NCH - 3, NCH - 2, NCH - 1):
        for d in sc_descs(t0):
            d.wait()
    plsc.subcore_barrier()
    pltpu.sync_copy(acc_sh.at[pl.ds(rows0, ZR)],
                    out_hbm.at[cid, pl.ds(rows0, ZR)])


def _sc_scatter(vals, i_idx, zrows, w, tiled, ebase, voff):
    mesh = plsc.VectorSubcoreMesh(core_axis_name="c", subcore_axis_name="s")
    return pl.kernel(
        functools.partial(_scatter_body, w, ebase, voff),
        compiler_params=pltpu.CompilerParams(use_tc_tiling_on_sc=tiled),
        out_type=jax.ShapeDtypeStruct((NC, NP, w), jnp.float32),
        mesh=mesh,
        scratch_types=[
            pltpu.VMEM((DP, CH), jnp.int32),
            pltpu.VMEM((DP, CH, w), jnp.float32),
            pltpu.VMEM_SHARED((NP, w), jnp.float32),
            pltpu.SemaphoreType.DMA((DP,)),
            pltpu.SemaphoreType.DMA((DP,)),
        ],
    )(vals, i_idx, zrows)


def _n1_body(h_ref, aggp0_ref, aggp1_ref, na_ref, x_ref, trp0_ref, trp1_ref,
             wh_ref, wa_ref, wn_ref, b_ref, z1h_ref, xnew_ref, ssum_ref,
             ssq_ref):
    agg = (aggp0_ref[0] + aggp0_ref[1]) + (aggp1_ref[0] + aggp1_ref[1])
    z1 = (jnp.dot(h_ref[...], wh_ref[...], preferred_element_type=jnp.float32)
          + jnp.dot(agg, wa_ref[...], preferred_element_type=jnp.float32)
          + jnp.dot(na_ref[...], wn_ref[...], preferred_element_type=jnp.float32)
          + b_ref[...])
    z1h_ref[...] = z1

    trp = (trp0_ref[0] + trp0_ref[1]) + (trp1_ref[0] + trp1_ref[1])
    num = trp[:, :4]
    cnt = trp[:, 4:5]
    xnew_ref[...] = x_ref[...] + num / jnp.maximum(cnt, 1.0)

    @pl.when(pl.program_id(0) == 0)
    def _():
        ssum_ref[...] = jnp.zeros_like(ssum_ref)
        ssq_ref[...] = jnp.zeros_like(ssq_ref)

    ssum_ref[...] += jnp.sum(z1, axis=0, keepdims=True)
    ssq_ref[...] += jnp.sum(z1 * z1, axis=0, keepdims=True)


def _n2_body(z1h_ref, g_ref, b_ref, wh2_ref, bh2_ref, h_ref, out_ref):
    zh = jnp.maximum(z1h_ref[...] * g_ref[...] + b_ref[...], 0.0)
    out_ref[...] = h_ref[...] + jnp.dot(
        zh, wh2_ref[...], preferred_element_type=jnp.float32) + bh2_ref[...]


def kernel(h, x, node_attr, W_e1, gamma_e, beta_e, W_e2, b_e2, W_m, b_m,
           W_h1, b_h1, gamma_h, beta_h, W_h2, b_h2, W_x1, b_x1, W_x2, edges):
    i_idx = edges[0]
    j_idx = edges[1]
    wnd = W_e1[2 * D:]

    ta, tb = pl.pallas_call(
        _ab_body,
        grid=(GRID_N,),
        in_specs=[
            pl.BlockSpec((BN, D), lambda e: (e, 0)),
            pl.BlockSpec((BN, 4), lambda e: (e, 0)),
            pl.BlockSpec((D, H), lambda e: (0, 0)),
            pl.BlockSpec((D, H), lambda e: (0, 0)),
        ],
        out_specs=[
            pl.BlockSpec((BN, D), lambda e: (e, 0)),
            pl.BlockSpec((BN, D), lambda e: (e, 0)),
        ],
        out_shape=[
            jax.ShapeDtypeStruct((N, D), jnp.int32),
            jax.ShapeDtypeStruct((N, D), jnp.int32),
        ],
    )(h, x, W_e1[:D], W_e1[D:2 * D])

    gh = [_sc_gather(i_idx, j_idx, ta, tb, hh * E2) for hh in (0, 1)]

    e1_out = []
    for hh in (0, 1):
        gi, gj = gh[hh]
        e1_out.append(pl.pallas_call(
            _e1_body,
            grid=(GRID_E2,),
            in_specs=[
                pl.BlockSpec((BE, D), lambda e: (e, 0)),
                pl.BlockSpec((BE, D), lambda e: (e, 0)),
                pl.BlockSpec((2, H), lambda e: (0, 0)),
            ],
            out_specs=[
                pl.BlockSpec((BE, H), lambda e: (e, 0)),
                pl.BlockSpec((BE, 4), lambda e: (e, 0)),
                pl.BlockSpec((1, H), lambda e: (0, 0)),
                pl.BlockSpec((1, H), lambda e: (0, 0)),
            ],
            out_shape=[
                jax.ShapeDtypeStruct((E2, H), jnp.bfloat16),
                jax.ShapeDtypeStruct((E2, 4), jnp.float32),
                jax.ShapeDtypeStruct((1, H), jnp.float32),
                jax.ShapeDtypeStruct((1, H), jnp.float32),
            ],
        )(gi, gj, wnd))

    ssum = e1_out[0][2] + e1_out[1][2]
    ssq = e1_out[0][3] + e1_out[1][3]
    mu = ssum / E
    var = ssq / E - mu * mu
    ghat = gamma_e / jnp.sqrt(var[0] + 1e-5)
    bhat = beta_e - mu[0] * ghat

    e2_w = [W_e2, b_e2[None], W_m, b_m[None], W_x1, b_x1[None], W_x2]
    e2_wspecs = [
        pl.BlockSpec((H, H), lambda e: (0, 0)),
        pl.BlockSpec((1, H), lambda e: (0, 0)),
        pl.BlockSpec((H, 1), lambda e: (0, 0)),
        pl.BlockSpec((1, 1), lambda e: (0, 0)),
        pl.BlockSpec((H, H), lambda e: (0, 0)),
        pl.BlockSpec((1, H), lambda e: (0, 0)),
        pl.BlockSpec((H, 1), lambda e: (0, 0)),
    ]

    m0h, mf0, tr0 = pl.pallas_call(
        _e2_body,
        grid=(GRID_E2,),
        in_specs=[
            pl.BlockSpec((BE, H), lambda e: (e, 0)),
            pl.BlockSpec((BE, 4), lambda e: (e, 0)),
            pl.BlockSpec((1, H), lambda e: (0, 0)),
            pl.BlockSpec((1, H), lambda e: (0, 0)),
        ] + e2_wspecs,
        out_specs=[
            pl.BlockSpec((BE, H), lambda e: (e, 0)),
            pl.BlockSpec((BE, H), lambda e: (e, 0)),
            pl.BlockSpec((BE, 8), lambda e: (e, 0)),
        ],
        out_shape=[
            jax.ShapeDtypeStruct((E2, H), jnp.float32),
            jax.ShapeDtypeStruct((E, H), jnp.float32),
            jax.ShapeDtypeStruct((E2, 8), jnp.float32),
        ],
    )(e1_out[0][0], e1_out[0][1], ghat[None], bhat[None], *e2_w)

    m1h, m, tr1 = pl.pallas_call(
        _e2_body_alias,
        grid=(GRID_E2,),
        in_specs=[
            pl.BlockSpec((BE, H), lambda e: (e, 0)),
            pl.BlockSpec((BE, 4), lambda e: (e, 0)),
            pl.BlockSpec((1, H), lambda e: (0, 0)),
            pl.BlockSpec((1, H), lambda e: (0, 0)),
        ] + e2_wspecs + [pl.BlockSpec(memory_space=pl.ANY)],
        out_specs=[
            pl.BlockSpec((BE, H), lambda e: (e, 0)),
            pl.BlockSpec((BE, H), lambda e: (e + GRID_E2, 0)),
            pl.BlockSpec((BE, 8), lambda e: (e, 0)),
        ],
        out_shape=[
            jax.ShapeDtypeStruct((E2, H), jnp.float32),
            jax.ShapeDtypeStruct((E, H), jnp.float32),
            jax.ShapeDtypeStruct((E2, 8), jnp.float32),
        ],
        input_output_aliases={11: 1},
    )(e1_out[1][0], e1_out[1][1], ghat[None], bhat[None], *e2_w, mf0)

    zd = jnp.zeros((ZR, D), jnp.float32)
    z8 = jnp.zeros((ZR, 8), jnp.float32)
    aggp0 = _sc_scatter(m0h, i_idx, zd, D, True, 0, 0)
    aggp1 = _sc_scatter(m1h, i_idx, zd, D, True, E2, 0)
    trnp0 = _sc_scatter(tr0, i_idx, z8, 8, False, 0, 0)
    trnp1 = _sc_scatter(tr1, i_idx, z8, 8, False, E2, 0)
    aggp0, aggp1 = aggp0[:, :N], aggp1[:, :N]
    trnp0, trnp1 = trnp0[:, :N], trnp1[:, :N]

    z1h, x_new, nsum, nsq = pl.pallas_call(
        _n1_body,
        grid=(GRID_N,),
        in_specs=[
            pl.BlockSpec((BN, D), lambda e: (e, 0)),
            pl.BlockSpec((NC, BN, H), lambda e: (0, e, 0)),
            pl.BlockSpec((NC, BN, H), lambda e: (0, e, 0)),
            pl.BlockSpec((BN, NA), lambda e: (e, 0)),
            pl.BlockSpec((BN, 4), lambda e: (e, 0)),
            pl.BlockSpec((NC, BN, 8), lambda e: (0, e, 0)),
            pl.BlockSpec((NC, BN, 8), lambda e: (0, e, 0)),
            pl.BlockSpec((D, H), lambda e: (0, 0)),
            pl.BlockSpec((H, H), lambda e: (0, 0)),
            pl.BlockSpec((NA, H), lambda e: (0, 0)),
            pl.BlockSpec((1, H), lambda e: (0, 0)),
        ],
        out_specs=[
            pl.BlockSpec((BN, H), lambda e: (e, 0)),
            pl.BlockSpec((BN, 4), lambda e: (e, 0)),
            pl.BlockSpec((1, H), lambda e: (0, 0)),
            pl.BlockSpec((1, H), lambda e: (0, 0)),
        ],
        out_shape=[
            jax.ShapeDtypeStruct((N, H), jnp.float32),
            jax.ShapeDtypeStruct((N, 4), jnp.float32),
            jax.ShapeDtypeStruct((1, H), jnp.float32),
            jax.ShapeDtypeStruct((1, H), jnp.float32),
        ],
    )(h, aggp0, aggp1, node_attr, x, trnp0, trnp1, W_h1[:D], W_h1[D:D + H],
      W_h1[D + H:], b_h1[None])

    mu_h = nsum / N
    var_h = nsq / N - mu_h * mu_h
    ghat_h = gamma_h / jnp.sqrt(var_h[0] + 1e-5)
    bhat_h = beta_h - mu_h[0] * ghat_h

    h_new = pl.pallas_call(
        _n2_body,
        grid=(GRID_N,),
        in_specs=[
            pl.BlockSpec((BN, H), lambda e: (e, 0)),
            pl.BlockSpec((1, H), lambda e: (0, 0)),
            pl.BlockSpec((1, H), lambda e: (0, 0)),
            pl.BlockSpec((H, D), lambda e: (0, 0)),
            pl.BlockSpec((1, D), lambda e: (0, 0)),
            pl.BlockSpec((BN, D), lambda e: (e, 0)),
        ],
        out_specs=pl.BlockSpec((BN, D), lambda e: (e, 0)),
        out_shape=jax.ShapeDtypeStruct((N, D), jnp.float32),
    )(z1h, ghat_h[None], bhat_h[None], W_h2, b_h2[None], h)

    return (h_new, x_new, m)


# speedup vs baseline: 1.0755x; 1.0755x over previous
import functools

import jax
import jax.numpy as jnp
from jax import lax
from jax.experimental import pallas as pl
from jax.experimental.pallas import tpu as pltpu
from jax.experimental.pallas import tpu_sc as plsc

N = 10000
E = 320000
D = 128
H = 128
NA = 16

NC = 2
NS = 16
NW = NC * NS
E2 = E // 2
EW = E2 // NW
CH = 40
NCH = EW // CH
EWS = E // NW
CHS = 80
NCHS = EWS // CHS
DP = 4
NP = 10240
ZR = NP // NS

BE = 2000
GRID_E2 = E2 // BE
BN = 2000
GRID_N = N // BN


def _psi(p):
    return jnp.sign(p) * jnp.log(jnp.abs(p) + 1)


def _pack2(lo_f32, hi_f32):
    lo = lax.bitcast_convert_type(lo_f32.astype(jnp.bfloat16),
                                  jnp.uint16).astype(jnp.uint32)
    hi = lax.bitcast_convert_type(hi_f32.astype(jnp.bfloat16),
                                  jnp.uint16).astype(jnp.uint32)
    return lax.bitcast_convert_type(lo | (hi << 16), jnp.int32)


def _ab_body(h_ref, x_ref, wa_ref, wb_ref, ta_ref, tb_ref):
    a = jnp.dot(h_ref[...], wa_ref[...], preferred_element_type=jnp.float32)
    b = jnp.dot(h_ref[...], wb_ref[...], preferred_element_type=jnp.float32)
    x = x_ref[...]
    xpad = jnp.concatenate([x, jnp.zeros((x.shape[0], D - 4), jnp.float32)],
                           axis=1)
    ta_ref[...] = _pack2(a, xpad)
    tb_ref[...] = _pack2(b, xpad)


def _gather_body(ebase, i_hbm, j_hbm, ta_hbm, tb_hbm, gi_out, gj_out,
                 idxi, idxj, bufa, bufb, isem, gsem, wsem):
    wid = lax.axis_index("s") * NC + lax.axis_index("c")
    base = ebase + wid * EW

    pltpu.make_async_copy(i_hbm.at[pl.ds(base, EW)], idxi, isem).start()
    pltpu.make_async_copy(j_hbm.at[pl.ds(base, EW)], idxj, isem).start()
    pltpu.make_async_copy(i_hbm.at[pl.ds(base, EW)], idxi, isem).wait()
    pltpu.make_async_copy(j_hbm.at[pl.ds(base, EW)], idxj, isem).wait()

    def gat_descs(t):
        b = lax.rem(t, DP)
        o = t * CH
        return [
            pltpu.make_async_copy(ta_hbm.at[idxi.at[pl.ds(o, CH)]],
                                  bufa.at[b], gsem.at[b]),
            pltpu.make_async_copy(tb_hbm.at[idxj.at[pl.ds(o, CH)]],
                                  bufb.at[b], gsem.at[b]),
        ]

    obase = wid * EW

    def wr_descs(t):
        b = lax.rem(t, DP)
        s = obase + t * CH
        return [
            pltpu.make_async_copy(bufa.at[b], gi_out.at[pl.ds(s, CH)], wsem.at[b]),
            pltpu.make_async_copy(bufb.at[b], gj_out.at[pl.ds(s, CH)], wsem.at[b]),
        ]

    def fire(descs):
        for d in descs:
            d.start()

    def drain(descs):
        for d in descs:
            d.wait()

    fire(gat_descs(0))
    fire(gat_descs(1))
    fire(gat_descs(2))

    def body(t, carry):
        drain(gat_descs(t))
        fire(wr_descs(t))

        @pl.when(t + 3 < NCH)
        def _():
            @pl.when(t >= 1)
            def _():
                drain(wr_descs(t - 1))

            fire(gat_descs(t + 3))

        return carry

    lax.fori_loop(0, NCH, body, 0)
    drain(wr_descs(NCH - 4))
    drain(wr_descs(NCH - 3))
    drain(wr_descs(NCH - 2))
    drain(wr_descs(NCH - 1))


def _sc_gather(i_idx, j_idx, ta, tb, ebase):
    mesh = plsc.VectorSubcoreMesh(core_axis_name="c", subcore_axis_name="s")
    return pl.kernel(
        functools.partial(_gather_body, ebase),
        out_type=[
            jax.ShapeDtypeStruct((E2, D), jnp.int32),
            jax.ShapeDtypeStruct((E2, D), jnp.int32),
        ],
        mesh=mesh,
        scratch_types=[
            pltpu.VMEM((EW,), jnp.int32),
            pltpu.VMEM((EW,), jnp.int32),
            pltpu.VMEM((DP, CH, D), jnp.int32),
            pltpu.VMEM((DP, CH, D), jnp.int32),
            pltpu.SemaphoreType.DMA,
            pltpu.SemaphoreType.DMA((DP,)),
            pltpu.SemaphoreType.DMA((DP,)),
        ],
    )(i_idx, j_idx, ta, tb)


def _unpack_lo(v_u32):
    return lax.bitcast_convert_type(v_u32 << 16, jnp.float32)


def _unpack_hi(v_u32):
    return lax.bitcast_convert_type(v_u32 & jnp.uint32(0xFFFF0000),
                                    jnp.float32)


def _e1_body(gi_ref, gj_ref, wnd_ref, z1_ref, xd_ref, ssum_ref, ssq_ref):
    gi = lax.bitcast_convert_type(gi_ref[...], jnp.uint32)
    gj = lax.bitcast_convert_type(gj_ref[...], jnp.uint32)
    ai = _unpack_lo(gi)
    bj = _unpack_lo(gj)
    xi = _unpack_hi(gi[:, :4])
    xj = _unpack_hi(gj[:, :4])
    xd = xi - xj
    psq = xd * xd
    norms = _psi(2.0 * psq[:, :1] - jnp.sum(psq, axis=1, keepdims=True))
    dsq = xi * xj
    dots = _psi(2.0 * dsq[:, :1] - jnp.sum(dsq, axis=1, keepdims=True))
    wnd = wnd_ref[...]
    z1 = ai + bj + norms * wnd[:1] + dots * wnd[1:2]
    z1_ref[...] = z1.astype(jnp.bfloat16)
    xd_ref[...] = xd

    @pl.when(pl.program_id(0) == 0)
    def _():
        ssum_ref[...] = jnp.zeros_like(ssum_ref)
        ssq_ref[...] = jnp.zeros_like(ssq_ref)

    ssum_ref[...] += jnp.sum(z1, axis=0, keepdims=True)
    ssq_ref[...] += jnp.sum(z1 * z1, axis=0, keepdims=True)


def _e2_body(z1_ref, xd_ref, g_ref, b_ref, we2_ref, be2_ref, wm_ref,
             bm_ref, wx1_ref, bx1_ref, wx2_ref, m_ref, tr_ref):
    z1 = z1_ref[...].astype(jnp.float32)
    z = jnp.maximum(z1 * g_ref[...] + b_ref[...], 0.0)
    out = jnp.maximum(
        jnp.dot(z.astype(jnp.bfloat16), we2_ref[...].astype(jnp.bfloat16),
                preferred_element_type=jnp.float32)
        + be2_ref[...], 0.0)
    w = jax.nn.sigmoid(
        jnp.dot(out, wm_ref[...], preferred_element_type=jnp.float32)
        + bm_ref[...])
    m = out * w
    m_ref[...] = m
    t = jnp.maximum(
        jnp.dot(m.astype(jnp.bfloat16), wx1_ref[...].astype(jnp.bfloat16),
                preferred_element_type=jnp.float32)
        + bx1_ref[...], 0.0)
    phix = jnp.dot(t, wx2_ref[...], preferred_element_type=jnp.float32)
    trans = jnp.clip(xd_ref[...] * phix, -100.0, 100.0)
    rows = trans.shape[0]
    tr_ref[...] = jnp.concatenate(
        [trans, jnp.ones((rows, 1), jnp.float32),
         jnp.zeros((rows, 3), jnp.float32)], axis=1)


def _e2_body_alias(z1_ref, xd_ref, g_ref, b_ref, we2_ref, be2_ref, wm_ref,
                   bm_ref, wx1_ref, bx1_ref, wx2_ref, mprev_ref, m_ref,
                   tr_ref):
    del mprev_ref
    _e2_body(z1_ref, xd_ref, g_ref, b_ref, we2_ref, be2_ref, wm_ref,
             bm_ref, wx1_ref, bx1_ref, wx2_ref, m_ref, tr_ref)


def _scatter_loop(i_hbm, v_hbm, base, vbase, idx, vbuf, acc_sh, lsem, ssem):
    def ld_descs(t):
        b = lax.rem(t, DP)
        s = base + t * CHS
        sv = vbase + t * CHS
        return [
            pltpu.make_async_copy(i_hbm.at[pl.ds(s, CHS)], idx.at[b], lsem.at[b]),
            pltpu.make_async_copy(v_hbm.at[pl.ds(sv, CHS)], vbuf.at[b], lsem.at[b]),
        ]

    def sc_descs(t):
        b = lax.rem(t, DP)
        return [
            pltpu.make_async_copy(vbuf.at[b], acc_sh.at[idx.at[b]], ssem.at[b]),
        ]

    for t0 in (0, 1, 2):
        for d in ld_descs(t0):
            d.start()

    def body(t, carry):
        for d in ld_descs(t):
            d.wait()
        for d in sc_descs(t):
            d.start(add=True)

        @pl.when(t + 3 < NCHS)
        def _():
            @pl.when(t >= 1)
            def _():
                for d in sc_descs(t - 1):
                    d.wait()

            for d in ld_descs(t + 3):
                d.start()

        return carry

    lax.fori_loop(0, NCHS, body, 0)
    for t0 in (NCHS - 4, NCHS - 3, NCHS - 2, NCHS - 1):
        for d in sc_descs(t0):
            d.wait()


def _scatter_m_body(v_hbm, i_hbm, zrows_hbm, out_hbm,
                    idx, vbuf, acc_sh, lsem, ssem):
    cid = lax.axis_index("c")
    sid = lax.axis_index("s")
    wid = sid * NC + cid
    rows0 = sid * ZR

    pltpu.sync_copy(zrows_hbm, acc_sh.at[pl.ds(rows0, ZR)])
    plsc.subcore_barrier()
    _scatter_loop(i_hbm, v_hbm, wid * EWS, wid * EWS, idx, vbuf, acc_sh,
                  lsem, ssem)
    plsc.subcore_barrier()
    pltpu.sync_copy(acc_sh.at[pl.ds(rows0, ZR)],
                    out_hbm.at[cid, pl.ds(rows0, ZR)])


def _scatter_tr_body(v0_hbm, v1_hbm, i_hbm, zrows_hbm, out_hbm,
                     idx, vbuf, acc_sh, lsem, ssem):
    cid = lax.axis_index("c")
    sid = lax.axis_index("s")
    wid = sid * NC + cid
    rows0 = sid * ZR

    pltpu.sync_copy(zrows_hbm, acc_sh.at[pl.ds(rows0, ZR)])
    plsc.subcore_barrier()

    @pl.when(wid < NW // 2)
    def _():
        _scatter_loop(i_hbm, v0_hbm, wid * EWS, wid * EWS, idx, vbuf,
                      acc_sh, lsem, ssem)

    @pl.when(wid >= NW // 2)
    def _():
        _scatter_loop(i_hbm, v1_hbm, wid * EWS, wid * EWS - E2, idx, vbuf,
                      acc_sh, lsem, ssem)

    plsc.subcore_barrier()
    pltpu.sync_copy(acc_sh.at[pl.ds(rows0, ZR)],
                    out_hbm.at[cid, pl.ds(rows0, ZR)])


def _sc_scatter_m(vals, i_idx, zrows):
    mesh = plsc.VectorSubcoreMesh(core_axis_name="c", subcore_axis_name="s")
    return pl.kernel(
        _scatter_m_body,
        out_type=jax.ShapeDtypeStruct((NC, NP, D), jnp.float32),
        mesh=mesh,
        scratch_types=[
            pltpu.VMEM((DP, CHS), jnp.int32),
            pltpu.VMEM((DP, CHS, D), jnp.float32),
            pltpu.VMEM_SHARED((NP, D), jnp.float32),
            pltpu.SemaphoreType.DMA((DP,)),
            pltpu.SemaphoreType.DMA((DP,)),
        ],
    )(vals, i_idx, zrows)


def _sc_scatter_tr(v0, v1, i_idx, zrows):
    mesh = plsc.VectorSubcoreMesh(core_axis_name="c", subcore_axis_name="s")
    return pl.kernel(
        _scatter_tr_body,
        compiler_params=pltpu.CompilerParams(use_tc_tiling_on_sc=False),
        out_type=jax.ShapeDtypeStruct((NC, NP, 8), jnp.float32),
        mesh=mesh,
        scratch_types=[
            pltpu.VMEM((DP, CHS), jnp.int32),
            pltpu.VMEM((DP, CHS, 8), jnp.float32),
            pltpu.VMEM_SHARED((NP, 8), jnp.float32),
            pltpu.SemaphoreType.DMA((DP,)),
            pltpu.SemaphoreType.DMA((DP,)),
        ],
    )(v0, v1, i_idx, zrows)


def _n1_body(h_ref, aggp_ref, na_ref, x_ref, trp_ref, wh_ref, wa_ref,
             wn_ref, b_ref, z1h_ref, xnew_ref, ssum_ref, ssq_ref):
    agg = aggp_ref[0] + aggp_ref[1]
    z1 = (jnp.dot(h_ref[...], wh_ref[...], preferred_element_type=jnp.float32)
          + jnp.dot(agg, wa_ref[...], preferred_element_type=jnp.float32)
          + jnp.dot(na_ref[...], wn_ref[...], preferred_element_type=jnp.float32)
          + b_ref[...])
    z1h_ref[...] = z1

    trp = trp_ref[0] + trp_ref[1]
    num = trp[:, :4]
    cnt = trp[:, 4:5]
    xnew_ref[...] = x_ref[...] + num / jnp.maximum(cnt, 1.0)

    @pl.when(pl.program_id(0) == 0)
    def _():
        ssum_ref[...] = jnp.zeros_like(ssum_ref)
        ssq_ref[...] = jnp.zeros_like(ssq_ref)

    ssum_ref[...] += jnp.sum(z1, axis=0, keepdims=True)
    ssq_ref[...] += jnp.sum(z1 * z1, axis=0, keepdims=True)


def _n2_body(z1h_ref, g_ref, b_ref, wh2_ref, bh2_ref, h_ref, out_ref):
    zh = jnp.maximum(z1h_ref[...] * g_ref[...] + b_ref[...], 0.0)
    out_ref[...] = h_ref[...] + jnp.dot(
        zh, wh2_ref[...], preferred_element_type=jnp.float32) + bh2_ref[...]


def kernel(h, x, node_attr, W_e1, gamma_e, beta_e, W_e2, b_e2, W_m, b_m,
           W_h1, b_h1, gamma_h, beta_h, W_h2, b_h2, W_x1, b_x1, W_x2, edges):
    i_idx = edges[0]
    j_idx = edges[1]
    wnd = W_e1[2 * D:]

    ta, tb = pl.pallas_call(
        _ab_body,
        grid=(GRID_N,),
        in_specs=[
            pl.BlockSpec((BN, D), lambda e: (e, 0)),
            pl.BlockSpec((BN, 4), lambda e: (e, 0)),
            pl.BlockSpec((D, H), lambda e: (0, 0)),
            pl.BlockSpec((D, H), lambda e: (0, 0)),
        ],
        out_specs=[
            pl.BlockSpec((BN, D), lambda e: (e, 0)),
            pl.BlockSpec((BN, D), lambda e: (e, 0)),
        ],
        out_shape=[
            jax.ShapeDtypeStruct((N, D), jnp.int32),
            jax.ShapeDtypeStruct((N, D), jnp.int32),
        ],
    )(h, x, W_e1[:D], W_e1[D:2 * D])

    gh = [_sc_gather(i_idx, j_idx, ta, tb, hh * E2) for hh in (0, 1)]

    e1_out = []
    for hh in (0, 1):
        gi, gj = gh[hh]
        e1_out.append(pl.pallas_call(
            _e1_body,
            grid=(GRID_E2,),
            in_specs=[
                pl.BlockSpec((BE, D), lambda e: (e, 0)),
                pl.BlockSpec((BE, D), lambda e: (e, 0)),
                pl.BlockSpec((2, H), lambda e: (0, 0)),
            ],
            out_specs=[
                pl.BlockSpec((BE, H), lambda e: (e, 0)),
                pl.BlockSpec((BE, 4), lambda e: (e, 0)),
                pl.BlockSpec((1, H), lambda e: (0, 0)),
                pl.BlockSpec((1, H), lambda e: (0, 0)),
            ],
            out_shape=[
                jax.ShapeDtypeStruct((E2, H), jnp.bfloat16),
                jax.ShapeDtypeStruct((E2, 4), jnp.float32),
                jax.ShapeDtypeStruct((1, H), jnp.float32),
                jax.ShapeDtypeStruct((1, H), jnp.float32),
            ],
        )(gi, gj, wnd))

    ssum = e1_out[0][2] + e1_out[1][2]
    ssq = e1_out[0][3] + e1_out[1][3]
    mu = ssum / E
    var = ssq / E - mu * mu
    ghat = gamma_e / jnp.sqrt(var[0] + 1e-5)
    bhat = beta_e - mu[0] * ghat

    e2_w = [W_e2, b_e2[None], W_m, b_m[None], W_x1, b_x1[None], W_x2]
    e2_wspecs = [
        pl.BlockSpec((H, H), lambda e: (0, 0)),
        pl.BlockSpec((1, H), lambda e: (0, 0)),
        pl.BlockSpec((H, 1), lambda e: (0, 0)),
        pl.BlockSpec((1, 1), lambda e: (0, 0)),
        pl.BlockSpec((H, H), lambda e: (0, 0)),
        pl.BlockSpec((1, H), lambda e: (0, 0)),
        pl.BlockSpec((H, 1), lambda e: (0, 0)),
    ]

    m0, tr0 = pl.pallas_call(
        _e2_body,
        grid=(GRID_E2,),
        in_specs=[
            pl.BlockSpec((BE, H), lambda e: (e, 0)),
            pl.BlockSpec((BE, 4), lambda e: (e, 0)),
            pl.BlockSpec((1, H), lambda e: (0, 0)),
            pl.BlockSpec((1, H), lambda e: (0, 0)),
        ] + e2_wspecs,
        out_specs=[
            pl.BlockSpec((BE, H), lambda e: (e, 0)),
            pl.BlockSpec((BE, 8), lambda e: (e, 0)),
        ],
        out_shape=[
            jax.ShapeDtypeStruct((E, H), jnp.float32),
            jax.ShapeDtypeStruct((E2, 8), jnp.float32),
        ],
    )(e1_out[0][0], e1_out[0][1], ghat[None], bhat[None], *e2_w)

    m, tr1 = pl.pallas_call(
        _e2_body_alias,
        grid=(GRID_E2,),
        in_specs=[
            pl.BlockSpec((BE, H), lambda e: (e, 0)),
            pl.BlockSpec((BE, 4), lambda e: (e, 0)),
            pl.BlockSpec((1, H), lambda e: (0, 0)),
            pl.BlockSpec((1, H), lambda e: (0, 0)),
        ] + e2_wspecs + [pl.BlockSpec(memory_space=pl.ANY)],
        out_specs=[
            pl.BlockSpec((BE, H), lambda e: (e + GRID_E2, 0)),
            pl.BlockSpec((BE, 8), lambda e: (e, 0)),
        ],
        out_shape=[
            jax.ShapeDtypeStruct((E, H), jnp.float32),
            jax.ShapeDtypeStruct((E2, 8), jnp.float32),
        ],
        input_output_aliases={11: 0},
    )(e1_out[1][0], e1_out[1][1], ghat[None], bhat[None], *e2_w, m0)

    zd = jnp.zeros((ZR, D), jnp.float32)
    z8 = jnp.zeros((ZR, 8), jnp.float32)
    aggp = _sc_scatter_m(m, i_idx, zd)
    trnp = _sc_scatter_tr(tr0, tr1, i_idx, z8)

    z1h, x_new, nsum, nsq = pl.pallas_call(
        _n1_body,
        grid=(GRID_N,),
        in_specs=[
            pl.BlockSpec((BN, D), lambda e: (e, 0)),
            pl.BlockSpec((NC, BN, H), lambda e: (0, e, 0)),
            pl.BlockSpec((BN, NA), lambda e: (e, 0)),
            pl.BlockSpec((BN, 4), lambda e: (e, 0)),
            pl.BlockSpec((NC, BN, 8), lambda e: (0, e, 0)),
            pl.BlockSpec((D, H), lambda e: (0, 0)),
            pl.BlockSpec((H, H), lambda e: (0, 0)),
            pl.BlockSpec((NA, H), lambda e: (0, 0)),
            pl.BlockSpec((1, H), lambda e: (0, 0)),
        ],
        out_specs=[
            pl.BlockSpec((BN, H), lambda e: (e, 0)),
            pl.BlockSpec((BN, 4), lambda e: (e, 0)),
            pl.BlockSpec((1, H), lambda e: (0, 0)),
            pl.BlockSpec((1, H), lambda e: (0, 0)),
        ],
        out_shape=[
            jax.ShapeDtypeStruct((N, H), jnp.float32),
            jax.ShapeDtypeStruct((N, 4), jnp.float32),
            jax.ShapeDtypeStruct((1, H), jnp.float32),
            jax.ShapeDtypeStruct((1, H), jnp.float32),
        ],
    )(h, aggp, node_attr, x, trnp, W_h1[:D], W_h1[D:D + H],
      W_h1[D + H:], b_h1[None])

    mu_h = nsum / N
    var_h = nsq / N - mu_h * mu_h
    ghat_h = gamma_h / jnp.sqrt(var_h[0] + 1e-5)
    bhat_h = beta_h - mu_h[0] * ghat_h

    h_new = pl.pallas_call(
        _n2_body,
        grid=(GRID_N,),
        in_specs=[
            pl.BlockSpec((BN, H), lambda e: (e, 0)),
            pl.BlockSpec((1, H), lambda e: (0, 0)),
            pl.BlockSpec((1, H), lambda e: (0, 0)),
            pl.BlockSpec((H, D), lambda e: (0, 0)),
            pl.BlockSpec((1, D), lambda e: (0, 0)),
            pl.BlockSpec((BN, D), lambda e: (e, 0)),
        ],
        out_specs=pl.BlockSpec((BN, D), lambda e: (e, 0)),
        out_shape=jax.ShapeDtypeStruct((N, D), jnp.float32),
    )(z1h, ghat_h[None], bhat_h[None], W_h2, b_h2[None], h)

    return (h_new, x_new, m)
